# baseline (speedup 1.0000x reference)
import functools

import jax
import jax.numpy as jnp
from jax.experimental import pallas as pl
from jax.experimental.pallas import tpu as pltpu

D_MODEL = 1024
N_IDX = 4 * 8192
RPS = 8


def _tc_body(idx_ref, *refs):
    in_refs = refs[:RPS]
    out_ref = refs[RPS]
    for j in range(RPS):
        out_ref[j, 0, :] = in_refs[j][0, 0, :]


def _tc_gather(t_flat, pe):
    n = t_flat.shape[0]
    grid = (n // RPS,)
    in_specs = [
        pl.BlockSpec((1, 1, D_MODEL), functools.partial(
            lambda j, i, idx: (idx[i * RPS + j], 0, 0), j))
        for j in range(RPS)
    ]
    return pl.pallas_call(
        _tc_body,
        grid_spec=pltpu.PrefetchScalarGridSpec(
            num_scalar_prefetch=1,
            grid=grid,
            in_specs=in_specs,
            out_specs=pl.BlockSpec((RPS, 1, D_MODEL), lambda i, idx: (i, 0, 0)),
        ),
        out_shape=jax.ShapeDtypeStruct((n, 1, D_MODEL), jnp.float32),
    )(t_flat, *([pe.reshape(pe.shape[0], 1, D_MODEL)] * RPS))


@jax.jit
def kernel(t, pe):
    t_flat = t.reshape(-1)
    out = _tc_gather(t_flat, pe)[:, 0, :]
    return out.reshape(t.shape + (D_MODEL,))


# speedup vs baseline: 18.9740x; 18.9740x over previous
import functools

import jax
import jax.numpy as jnp
from jax import lax
from jax.experimental import pallas as pl
from jax.experimental.pallas import tpu as pltpu
from jax.experimental.pallas import tpu_sc as plsc

D_MODEL = 1024
N_IDX = 4 * 8192

_info = plsc.get_sparse_core_info()
NC, NS = _info.num_cores, _info.num_subcores
NW = NC * NS
B_PER_W = N_IDX // NW
CHUNK = 16
NBUF = 4
N_CHUNK = B_PER_W // CHUNK
LOOK = 2


def _gather_body(t_hbm, pe_hbm, out_hbm, idx_v, *rest):
    bufs = rest[:NBUF]
    g_sems = rest[NBUF:2 * NBUF]
    w_sems = rest[2 * NBUF:]
    wid = lax.axis_index("s") * NC + lax.axis_index("c")
    base = wid * B_PER_W
    pltpu.sync_copy(t_hbm.at[pl.ds(base, B_PER_W)], idx_v)

    def start_gather(g_off, b):
        pltpu.async_copy(
            pe_hbm.at[idx_v.at[pl.ds(g_off, CHUNK)]], bufs[b], g_sems[b])

    def wait_gather(b):
        pltpu.make_async_copy(
            pe_hbm.at[pl.ds(0, CHUNK)], bufs[b], g_sems[b]).wait()

    def start_write(g_off, b):
        pltpu.async_copy(
            bufs[b], out_hbm.at[pl.ds(base + g_off, CHUNK)], w_sems[b])

    def wait_write(b):
        pltpu.make_async_copy(
            bufs[b], out_hbm.at[pl.ds(base, CHUNK)], w_sems[b]).wait()

    for g in range(LOOK):
        start_gather(g * CHUNK, g % NBUF)
    for g in range(LOOK):
        b = g % NBUF
        wait_gather(b)
        start_write(g * CHUNK, b)
        start_gather((g + LOOK) * CHUNK, (g + LOOK) % NBUF)

    def step(i, carry):
        for k in range(NBUF):
            g = LOOK + i * NBUF + k
            b = (LOOK + k) % NBUF
            off = g * CHUNK
            wait_gather(b)
            start_write(off, b)
            bn = (b + LOOK) % NBUF
            wait_write(bn)
            start_gather(off + LOOK * CHUNK, bn)
        return carry

    n_main = N_CHUNK - 2 * LOOK
    assert n_main % NBUF == 0
    lax.fori_loop(0, n_main // NBUF, step, 0)

    for g in range(N_CHUNK - LOOK, N_CHUNK):
        b = g % NBUF
        wait_gather(b)
        start_write(g * CHUNK, b)
    for b in range(NBUF):
        wait_write(b)


@jax.jit
def kernel(t, pe):
    t_flat = t.reshape(-1)
    grid_kernel = functools.partial(
        pl.kernel,
        mesh=plsc.VectorSubcoreMesh(core_axis_name="c", subcore_axis_name="s"),
        out_type=jax.ShapeDtypeStruct((N_IDX, D_MODEL), jnp.float32),
        scratch_types=(
            [pltpu.VMEM((B_PER_W,), jnp.int32)]
            + [pltpu.VMEM((CHUNK, D_MODEL), jnp.float32)] * NBUF
            + [pltpu.SemaphoreType.DMA] * (2 * NBUF)
        ),
    )
    out = grid_kernel(_gather_body)(t_flat, pe)
    return out.reshape(t.shape + (D_MODEL,))
